# baseline (speedup 1.0000x reference)
import jax
import jax.numpy as jnp
from jax import lax
from jax.experimental import pallas as pl
from jax.experimental.pallas import tpu as pltpu
from jax.experimental.pallas import tpu_sc as plsc

NUM_REL = 6
NUM_HEADS = 32
SEQ = 64
LANES = 16


def _sc_relation_bias(w, idx):
    mesh = plsc.VectorSubcoreMesh(core_axis_name="c", subcore_axis_name="s")

    def body(w_hbm, idx_hbm, out_hbm, w_v, idx_v, out_v, sem_w, sem_i):
        h = lax.axis_index("s") * 2 + lax.axis_index("c")
        cw = pltpu.async_copy(w_hbm, w_v, sem_w)
        ci = pltpu.async_copy(idx_hbm, idx_v, sem_i)
        cw.wait()
        ci.wait()
        hvec = jnp.full((LANES,), h, dtype=jnp.int32)

        def row_body(r, carry):
            for c in range(SEQ // LANES):
                sl = pl.ds(c * LANES, LANES)
                iv = idx_v[r, sl]
                out_v[r, sl] = plsc.load_gather(w_v, [iv, hvec])
            return carry

        lax.fori_loop(0, SEQ, row_body, 0)
        pltpu.sync_copy(out_v, out_hbm.at[h])

    return pl.kernel(
        body,
        mesh=mesh,
        compiler_params=pltpu.CompilerParams(needs_layout_passes=False),
        out_type=jax.ShapeDtypeStruct((NUM_HEADS, SEQ, SEQ), jnp.float32),
        scratch_types=[
            pltpu.VMEM((NUM_REL, NUM_HEADS), jnp.float32),
            pltpu.VMEM((SEQ, SEQ), jnp.int32),
            pltpu.VMEM((SEQ, SEQ), jnp.float32),
            pltpu.SemaphoreType.DMA,
            pltpu.SemaphoreType.DMA,
        ],
    )(w, idx)


def kernel(embedding_weight, relation_index):
    w = embedding_weight.astype(jnp.float32)
    idx = relation_index.astype(jnp.int32)
    return _sc_relation_bias(w, idx)


# speedup vs baseline: 1.0687x; 1.0687x over previous
import jax
import jax.numpy as jnp
from jax import lax
from jax.experimental import pallas as pl
from jax.experimental.pallas import tpu as pltpu
from jax.experimental.pallas import tpu_sc as plsc

NUM_REL = 6
NUM_HEADS = 32
SEQ = 64
LANES = 16


def _sc_relation_bias(w, idx):
    mesh = plsc.VectorSubcoreMesh(core_axis_name="c", subcore_axis_name="s")

    def body(w_hbm, idx_hbm, out_hbm, w_v, idx_v, out_v, wcol_v, sem_w, sem_i):
        h = lax.axis_index("s") * 2 + lax.axis_index("c")
        cw = pltpu.async_copy(w_hbm, w_v, sem_w)
        ci = pltpu.async_copy(idx_hbm, idx_v, sem_i)
        cw.wait()
        ci.wait()
        hvec = jnp.full((LANES,), h, dtype=jnp.int32)
        rvec = jnp.minimum(lax.iota(jnp.int32, LANES), NUM_REL - 1)
        wcol_v[...] = plsc.load_gather(w_v, [rvec, hvec])

        @plsc.parallel_loop(0, SEQ, step=1, unroll=8)
        def row_body(r):
            for c in range(SEQ // LANES):
                sl = pl.ds(c * LANES, LANES)
                out_v[r, sl] = plsc.load_gather(wcol_v, [idx_v[r, sl]])

        pltpu.sync_copy(out_v, out_hbm.at[h])

    return pl.kernel(
        body,
        mesh=mesh,
        compiler_params=pltpu.CompilerParams(needs_layout_passes=False),
        out_type=jax.ShapeDtypeStruct((NUM_HEADS, SEQ, SEQ), jnp.float32),
        scratch_types=[
            pltpu.VMEM((NUM_REL, NUM_HEADS), jnp.float32),
            pltpu.VMEM((SEQ, SEQ), jnp.int32),
            pltpu.VMEM((SEQ, SEQ), jnp.float32),
            pltpu.VMEM((LANES,), jnp.float32),
            pltpu.SemaphoreType.DMA,
            pltpu.SemaphoreType.DMA,
        ],
    )(w, idx)


def kernel(embedding_weight, relation_index):
    w = embedding_weight.astype(jnp.float32)
    idx = relation_index.astype(jnp.int32)
    return _sc_relation_bias(w, idx)


# speedup vs baseline: 1.0710x; 1.0022x over previous
import jax
import jax.numpy as jnp
from jax import lax
from jax.experimental import pallas as pl
from jax.experimental.pallas import tpu as pltpu
from jax.experimental.pallas import tpu_sc as plsc

NUM_REL = 6
NUM_HEADS = 32
SEQ = 64
LANES = 16
NH_TC = 24
NH_SC = NUM_HEADS - NH_TC
BANDS = 32 // NH_SC
BAND_ROWS = SEQ // BANDS


def _sc_part(w, idx):
    mesh = plsc.VectorSubcoreMesh(core_axis_name="c", subcore_axis_name="s")

    def body(w_hbm, idx_hbm, out_hbm, w_v, idx_v, out_v, wcol_v, sem_w, sem_i):
        wid = lax.axis_index("s") * 2 + lax.axis_index("c")
        hh = wid // BANDS
        rb = (wid % BANDS) * BAND_ROWS
        cw = pltpu.async_copy(w_hbm, w_v, sem_w)
        ci = pltpu.async_copy(idx_hbm.at[pl.ds(rb, BAND_ROWS)], idx_v, sem_i)
        cw.wait()
        ci.wait()
        hvec = jnp.full((LANES,), NH_TC + hh, dtype=jnp.int32)
        rvec = jnp.minimum(lax.iota(jnp.int32, LANES), NUM_REL - 1)
        wcol_v[...] = plsc.load_gather(w_v, [rvec, hvec])

        @plsc.parallel_loop(0, BAND_ROWS, step=1, unroll=8)
        def row_body(r):
            for c in range(SEQ // LANES):
                sl = pl.ds(c * LANES, LANES)
                out_v[r, sl] = plsc.load_gather(wcol_v, [idx_v[r, sl]])

        pltpu.sync_copy(out_v, out_hbm.at[hh, pl.ds(rb, BAND_ROWS)])

    return pl.kernel(
        body,
        mesh=mesh,
        compiler_params=pltpu.CompilerParams(needs_layout_passes=False),
        out_type=jax.ShapeDtypeStruct((NH_SC, SEQ, SEQ), jnp.float32),
        scratch_types=[
            pltpu.VMEM((NUM_REL, NUM_HEADS), jnp.float32),
            pltpu.VMEM((BAND_ROWS, SEQ), jnp.int32),
            pltpu.VMEM((BAND_ROWS, SEQ), jnp.float32),
            pltpu.VMEM((LANES,), jnp.float32),
            pltpu.SemaphoreType.DMA,
            pltpu.SemaphoreType.DMA,
        ],
    )(w, idx)


def _tc_body(w_ref, idx_ref, out_ref):
    idx = idx_ref[...]
    masks = [idx == r for r in range(1, NUM_REL)]
    for h in range(NH_TC):
        acc = jnp.full((SEQ, SEQ), w_ref[0, h], dtype=jnp.float32)
        for r in range(1, NUM_REL):
            acc = jnp.where(masks[r - 1], w_ref[r, h], acc)
        out_ref[h] = acc


def _tc_part(w, idx):
    return pl.pallas_call(
        _tc_body,
        out_shape=jax.ShapeDtypeStruct((NH_TC, SEQ, SEQ), jnp.float32),
        in_specs=[
            pl.BlockSpec(memory_space=pltpu.SMEM),
            pl.BlockSpec(memory_space=pltpu.VMEM),
        ],
        out_specs=pl.BlockSpec(memory_space=pltpu.VMEM),
    )(w, idx)


def kernel(embedding_weight, relation_index):
    w = embedding_weight.astype(jnp.float32)
    idx = relation_index.astype(jnp.int32)
    sc_out = _sc_part(w, idx)
    tc_out = _tc_part(w, idx)
    return jnp.concatenate([tc_out, sc_out], axis=0)


# speedup vs baseline: 1.1392x; 1.0637x over previous
import jax
import jax.numpy as jnp
from jax import lax
from jax.experimental import pallas as pl
from jax.experimental.pallas import tpu as pltpu
from jax.experimental.pallas import tpu_sc as plsc

NUM_REL = 6
NUM_HEADS = 32
SEQ = 64
LANES = 16
NW = 32
ROWS = SEQ // NW
CHUNKS = ROWS * SEQ // LANES


def _sc_relation_bias(w, idx):
    mesh = plsc.VectorSubcoreMesh(core_axis_name="c", subcore_axis_name="s")

    def body(w_hbm, idx_hbm, out_hbm, w_v, idx_v, out_v, wt_v, sem_w, sem_i):
        wid = lax.axis_index("s") * 2 + lax.axis_index("c")
        r0 = wid * ROWS
        cw = pltpu.async_copy(w_hbm, w_v, sem_w)
        ci = pltpu.async_copy(idx_hbm.at[pl.ds(r0, ROWS)], idx_v, sem_i)
        cw.wait()
        ci.wait()
        rvec = jnp.minimum(lax.iota(jnp.int32, LANES), NUM_REL - 1)
        for h in range(NUM_HEADS):
            hvec = jnp.full((LANES,), h, dtype=jnp.int32)
            wt_v[h] = plsc.load_gather(w_v, [rvec, hvec])
        chunks = [
            idx_v[c // (SEQ // LANES), pl.ds((c % (SEQ // LANES)) * LANES, LANES)]
            for c in range(CHUNKS)
        ]
        @plsc.parallel_loop(0, NUM_HEADS, step=1, unroll=8)
        def h_body(h):
            hvec = jnp.full((LANES,), h, dtype=jnp.int32)
            for c in range(CHUNKS):
                sl = pl.ds((c % (SEQ // LANES)) * LANES, LANES)
                out_v[h, c // (SEQ // LANES), sl] = plsc.load_gather(
                    wt_v, [hvec, chunks[c]]
                )
        pltpu.sync_copy(out_v, out_hbm.at[:, pl.ds(r0, ROWS), :])

    return pl.kernel(
        body,
        mesh=mesh,
        compiler_params=pltpu.CompilerParams(needs_layout_passes=False),
        out_type=jax.ShapeDtypeStruct((NUM_HEADS, SEQ, SEQ), jnp.float32),
        scratch_types=[
            pltpu.VMEM((NUM_REL, NUM_HEADS), jnp.float32),
            pltpu.VMEM((ROWS, SEQ), jnp.int32),
            pltpu.VMEM((NUM_HEADS, ROWS, SEQ), jnp.float32),
            pltpu.VMEM((NUM_HEADS, LANES), jnp.float32),
            pltpu.SemaphoreType.DMA,
            pltpu.SemaphoreType.DMA,
        ],
    )(w, idx)


def kernel(embedding_weight, relation_index):
    w = embedding_weight.astype(jnp.float32)
    idx = relation_index.astype(jnp.int32)
    return _sc_relation_bias(w, idx)
